# initial kernel scaffold (unmeasured)
import jax
import jax.numpy as jnp
from jax import lax
from jax.experimental import pallas as pl
from jax.experimental.pallas import tpu as pltpu

N_DEV = 4
AXIS = "i"

HID_BLK = 1024


def _neighbor_barrier(left, right):
    barrier = pltpu.get_barrier_semaphore()
    for nbr in (left, right):
        pl.semaphore_signal(
            barrier, inc=1, device_id=(nbr,),
            device_id_type=pl.DeviceIdType.MESH,
        )
    pl.semaphore_wait(barrier, 2)


def _ring_allgather(x_shard, collective_id):
    m, n = x_shard.shape

    def body(x_ref, out_ref, comm_ref, send_sems, recv_sems):
        my = lax.axis_index(AXIS)
        left = (my - 1) % N_DEV
        right = (my + 1) % N_DEV
        _neighbor_barrier(left, right)

        out_ref[pl.ds(my * m, m), :] = x_ref[:, :]
        comm_ref[0, :, :] = x_ref[:, :]
        for h in range(N_DEV - 1):
            rdma = pltpu.make_async_remote_copy(
                src_ref=comm_ref.at[h],
                dst_ref=comm_ref.at[h + 1],
                send_sem=send_sems.at[h],
                recv_sem=recv_sems.at[h + 1],
                device_id=(right,),
                device_id_type=pl.DeviceIdType.MESH,
            )
            rdma.start()
            rdma.wait()
            origin = (my - h - 1) % N_DEV
            out_ref[pl.ds(origin * m, m), :] = comm_ref[h + 1, :, :]

    return pl.pallas_call(
        body,
        out_shape=jax.ShapeDtypeStruct((N_DEV * m, n), x_shard.dtype),
        in_specs=[pl.BlockSpec(memory_space=pltpu.VMEM)],
        out_specs=pl.BlockSpec(memory_space=pltpu.VMEM),
        scratch_shapes=[
            pltpu.VMEM((N_DEV, m, n), x_shard.dtype),
            pltpu.SemaphoreType.DMA((N_DEV,)),
            pltpu.SemaphoreType.DMA((N_DEV,)),
        ],
        compiler_params=pltpu.CompilerParams(collective_id=collective_id),
    )(x_shard)


def _ring_reduce(partial, collective_id, scatter_only=False):
    M, n = partial.shape
    m = M // N_DEV
    n_hops = (N_DEV - 1) if scatter_only else 2 * (N_DEV - 1)

    def body(x_ref, out_ref, comm_ref, send_sems, recv_sems):
        my = lax.axis_index(AXIS)
        left = (my - 1) % N_DEV
        right = (my + 1) % N_DEV
        _neighbor_barrier(left, right)

        def hop(h):
            rdma = pltpu.make_async_remote_copy(
                src_ref=comm_ref.at[h],
                dst_ref=comm_ref.at[h + 1],
                send_sem=send_sems.at[h],
                recv_sem=recv_sems.at[h + 1],
                device_id=(right,),
                device_id_type=pl.DeviceIdType.MESH,
            )
            rdma.start()
            rdma.wait()

        c0 = (my + 3) % N_DEV
        comm_ref[0, :, :] = x_ref[pl.ds(c0 * m, m), :]
        for h in range(N_DEV - 1):
            hop(h)
            c = (my + 2 - h) % N_DEV
            comm_ref[h + 1, :, :] += x_ref[pl.ds(c * m, m), :]

        if scatter_only:
            out_ref[:, :] = comm_ref[N_DEV - 1, :, :]
        else:
            out_ref[pl.ds(my * m, m), :] = comm_ref[N_DEV - 1, :, :]
            for a in range(N_DEV - 1):
                h = (N_DEV - 1) + a
                hop(h)
                c = (my + N_DEV - 1 - a) % N_DEV
                out_ref[pl.ds(c * m, m), :] = comm_ref[h + 1, :, :]

    out_rows = m if scatter_only else M
    return pl.pallas_call(
        body,
        out_shape=jax.ShapeDtypeStruct((out_rows, n), partial.dtype),
        in_specs=[pl.BlockSpec(memory_space=pltpu.VMEM)],
        out_specs=pl.BlockSpec(memory_space=pltpu.VMEM),
        scratch_shapes=[
            pltpu.VMEM((n_hops + 1, m, n), partial.dtype),
            pltpu.SemaphoreType.DMA((n_hops + 1,)),
            pltpu.SemaphoreType.DMA((n_hops + 1,)),
        ],
        compiler_params=pltpu.CompilerParams(collective_id=collective_id),
    )(partial)


def _mlp_layer(x_full, Win, Wout):
    B, d = x_full.shape
    hl = Win.shape[1]
    n_blk = hl // HID_BLK

    def body(x_ref, win_ref, wout_ref, out_ref):
        @pl.when(pl.program_id(0) == 0)
        def _():
            out_ref[:, :] = jnp.zeros_like(out_ref)

        h = jnp.maximum(
            jnp.dot(x_ref[:, :], win_ref[:, :],
                    preferred_element_type=jnp.float32),
            0.0,
        )
        out_ref[:, :] += jnp.dot(
            h, wout_ref[:, :], preferred_element_type=jnp.float32
        )

    return pl.pallas_call(
        body,
        grid=(n_blk,),
        in_specs=[
            pl.BlockSpec((B, d), lambda c: (0, 0)),
            pl.BlockSpec((d, HID_BLK), lambda c: (0, c)),
            pl.BlockSpec((HID_BLK, d), lambda c: (c, 0)),
        ],
        out_specs=pl.BlockSpec((B, d), lambda c: (0, 0)),
        out_shape=jax.ShapeDtypeStruct((B, d), jnp.float32),
    )(x_full, Win, Wout)


def kernel(x, Win0, Wout0, Win1, Wout1, Win2, Wout2):
    xg = _ring_allgather(x, collective_id=0)
    p0 = _mlp_layer(xg, Win0, Wout0)
    x1 = _ring_reduce(p0, collective_id=1)
    p1 = _mlp_layer(x1, Win1, Wout1)
    x2 = _ring_reduce(p1, collective_id=2)
    p2 = _mlp_layer(x2, Win2, Wout2)
    return _ring_reduce(p2, collective_id=3, scatter_only=True)


# baseline (device time: 211016 ns/iter reference)
import jax
import jax.numpy as jnp
from jax import lax
from jax.experimental import pallas as pl
from jax.experimental.pallas import tpu as pltpu

N_DEV = 4
AXIS = "i"

HID_BLK = 512


def _neighbor_barrier(left, right):
    barrier = pltpu.get_barrier_semaphore()
    for nbr in (left, right):
        pl.semaphore_signal(
            barrier, inc=1, device_id=(nbr,),
            device_id_type=pl.DeviceIdType.MESH,
        )
    pl.semaphore_wait(barrier, 2)


def _ring_allgather(x_shard, collective_id):
    m, n = x_shard.shape

    def body(x_ref, out_ref, comm_ref, send_sems, recv_sems):
        my = lax.axis_index(AXIS)
        left = (my - 1) % N_DEV
        right = (my + 1) % N_DEV
        _neighbor_barrier(left, right)

        out_ref[pl.ds(my * m, m), :] = x_ref[:, :]
        comm_ref[0, :, :] = x_ref[:, :]
        for h in range(N_DEV - 1):
            rdma = pltpu.make_async_remote_copy(
                src_ref=comm_ref.at[h],
                dst_ref=comm_ref.at[h + 1],
                send_sem=send_sems.at[h],
                recv_sem=recv_sems.at[h + 1],
                device_id=(right,),
                device_id_type=pl.DeviceIdType.MESH,
            )
            rdma.start()
            rdma.wait()
            origin = (my - h - 1) % N_DEV
            out_ref[pl.ds(origin * m, m), :] = comm_ref[h + 1, :, :]

    return pl.pallas_call(
        body,
        out_shape=jax.ShapeDtypeStruct((N_DEV * m, n), x_shard.dtype),
        in_specs=[pl.BlockSpec(memory_space=pltpu.VMEM)],
        out_specs=pl.BlockSpec(memory_space=pltpu.VMEM),
        scratch_shapes=[
            pltpu.VMEM((N_DEV, m, n), x_shard.dtype),
            pltpu.SemaphoreType.DMA((N_DEV,)),
            pltpu.SemaphoreType.DMA((N_DEV,)),
        ],
        compiler_params=pltpu.CompilerParams(collective_id=collective_id),
    )(x_shard)


def _ring_reduce(partial, collective_id, scatter_only=False):
    M, n = partial.shape
    m = M // N_DEV
    n_hops = (N_DEV - 1) if scatter_only else 2 * (N_DEV - 1)

    def body(x_ref, out_ref, comm_ref, send_sems, recv_sems):
        my = lax.axis_index(AXIS)
        left = (my - 1) % N_DEV
        right = (my + 1) % N_DEV
        _neighbor_barrier(left, right)

        def hop(h):
            rdma = pltpu.make_async_remote_copy(
                src_ref=comm_ref.at[h],
                dst_ref=comm_ref.at[h + 1],
                send_sem=send_sems.at[h],
                recv_sem=recv_sems.at[h + 1],
                device_id=(right,),
                device_id_type=pl.DeviceIdType.MESH,
            )
            rdma.start()
            rdma.wait()

        c0 = (my + 3) % N_DEV
        comm_ref[0, :, :] = x_ref[pl.ds(c0 * m, m), :]
        for h in range(N_DEV - 1):
            hop(h)
            c = (my + 2 - h) % N_DEV
            comm_ref[h + 1, :, :] += x_ref[pl.ds(c * m, m), :]

        if scatter_only:
            out_ref[:, :] = comm_ref[N_DEV - 1, :, :]
        else:
            out_ref[pl.ds(my * m, m), :] = comm_ref[N_DEV - 1, :, :]
            for a in range(N_DEV - 1):
                h = (N_DEV - 1) + a
                hop(h)
                c = (my + N_DEV - 1 - a) % N_DEV
                out_ref[pl.ds(c * m, m), :] = comm_ref[h + 1, :, :]

    out_rows = m if scatter_only else M
    return pl.pallas_call(
        body,
        out_shape=jax.ShapeDtypeStruct((out_rows, n), partial.dtype),
        in_specs=[pl.BlockSpec(memory_space=pltpu.VMEM)],
        out_specs=pl.BlockSpec(memory_space=pltpu.VMEM),
        scratch_shapes=[
            pltpu.VMEM((n_hops + 1, m, n), partial.dtype),
            pltpu.SemaphoreType.DMA((n_hops + 1,)),
            pltpu.SemaphoreType.DMA((n_hops + 1,)),
        ],
        compiler_params=pltpu.CompilerParams(collective_id=collective_id),
    )(partial)


def _mlp_layer(x_full, Win, Wout):
    B, d = x_full.shape
    hl = Win.shape[1]
    n_blk = hl // HID_BLK

    def body(x_ref, win_ref, wout_ref, out_ref):
        @pl.when(pl.program_id(0) == 0)
        def _():
            out_ref[:, :] = jnp.zeros_like(out_ref)

        h = jnp.maximum(
            jnp.dot(x_ref[:, :], win_ref[:, :],
                    preferred_element_type=jnp.float32),
            0.0,
        )
        out_ref[:, :] += jnp.dot(
            h, wout_ref[:, :], preferred_element_type=jnp.float32
        )

    return pl.pallas_call(
        body,
        grid=(n_blk,),
        in_specs=[
            pl.BlockSpec((B, d), lambda c: (0, 0)),
            pl.BlockSpec((d, HID_BLK), lambda c: (0, c)),
            pl.BlockSpec((HID_BLK, d), lambda c: (c, 0)),
        ],
        out_specs=pl.BlockSpec((B, d), lambda c: (0, 0)),
        out_shape=jax.ShapeDtypeStruct((B, d), jnp.float32),
    )(x_full, Win, Wout)


def kernel(x, Win0, Wout0, Win1, Wout1, Win2, Wout2):
    xg = _ring_allgather(x, collective_id=0)
    p0 = _mlp_layer(xg, Win0, Wout0)
    x1 = _ring_reduce(p0, collective_id=1)
    p1 = _mlp_layer(x1, Win1, Wout1)
    x2 = _ring_reduce(p1, collective_id=2)
    p2 = _mlp_layer(x2, Win2, Wout2)
    return _ring_reduce(p2, collective_id=3, scatter_only=True)


# device time: 160005 ns/iter; 1.3188x vs baseline; 1.3188x over previous
import jax
import jax.numpy as jnp
from jax import lax
from jax.experimental import pallas as pl
from jax.experimental.pallas import tpu as pltpu

N_DEV = 4
AXIS = "i"
MESH_ID = pl.DeviceIdType.MESH

HID_BLK = 512


def _all_peer_barrier(my):
    barrier = pltpu.get_barrier_semaphore()
    for off in (1, 2, 3):
        pl.semaphore_signal(
            barrier, inc=1, device_id=((my + off) % N_DEV,),
            device_id_type=MESH_ID,
        )
    pl.semaphore_wait(barrier, N_DEV - 1)


def _direct_allgather(x_shard, collective_id):
    m, n = x_shard.shape

    def body(x_ref, out_ref, recv_sems, send_sems):
        my = lax.axis_index(AXIS)
        _all_peer_barrier(my)

        out_ref[pl.ds(my * m, m), :] = x_ref[:, :]
        sends = []
        for off in (1, 2, 3):
            t = (my + off) % N_DEV
            rdma = pltpu.make_async_remote_copy(
                src_ref=x_ref,
                dst_ref=out_ref.at[pl.ds(my * m, m), :],
                send_sem=send_sems.at[off - 1],
                recv_sem=recv_sems.at[3 - off],
                device_id=(t,),
                device_id_type=MESH_ID,
            )
            rdma.start()
            sends.append(rdma)
        for j in range(3):
            src_dev = (my + 1 + j) % N_DEV
            recv = pltpu.make_async_remote_copy(
                src_ref=x_ref,
                dst_ref=out_ref.at[pl.ds(src_dev * m, m), :],
                send_sem=send_sems.at[j],
                recv_sem=recv_sems.at[j],
                device_id=(src_dev,),
                device_id_type=MESH_ID,
            )
            recv.wait_recv()
        for rdma in sends:
            rdma.wait_send()

    return pl.pallas_call(
        body,
        out_shape=jax.ShapeDtypeStruct((N_DEV * m, n), x_shard.dtype),
        in_specs=[pl.BlockSpec(memory_space=pltpu.VMEM)],
        out_specs=pl.BlockSpec(memory_space=pltpu.VMEM),
        scratch_shapes=[
            pltpu.SemaphoreType.DMA((3,)),
            pltpu.SemaphoreType.DMA((3,)),
        ],
        compiler_params=pltpu.CompilerParams(collective_id=collective_id),
    )(x_shard)


def _direct_allreduce(partial, collective_id, scatter_only=False):
    M, n = partial.shape
    m = M // N_DEV

    def body(x_ref, out_ref, acc_ref, p_recv_sems, b_recv_sems, send_sems):
        my = lax.axis_index(AXIS)
        _all_peer_barrier(my)

        sends = []
        for off in (1, 2, 3):
            t = (my + off) % N_DEV
            rdma = pltpu.make_async_remote_copy(
                src_ref=x_ref.at[pl.ds(t * m, m), :],
                dst_ref=acc_ref.at[3 - off],
                send_sem=send_sems.at[off - 1],
                recv_sem=p_recv_sems.at[3 - off],
                device_id=(t,),
                device_id_type=MESH_ID,
            )
            rdma.start()
            sends.append(rdma)
        for j in range(3):
            recv = pltpu.make_async_remote_copy(
                src_ref=x_ref.at[pl.ds(0, m), :],
                dst_ref=acc_ref.at[j],
                send_sem=send_sems.at[j],
                recv_sem=p_recv_sems.at[j],
                device_id=(my,),
                device_id_type=MESH_ID,
            )
            recv.wait_recv()
        red = (
            x_ref[pl.ds(my * m, m), :]
            + acc_ref[0, :, :] + acc_ref[1, :, :] + acc_ref[2, :, :]
        )

        if scatter_only:
            out_ref[:, :] = red
        else:
            out_ref[pl.ds(my * m, m), :] = red
            for off in (1, 2, 3):
                t = (my + off) % N_DEV
                rdma = pltpu.make_async_remote_copy(
                    src_ref=out_ref.at[pl.ds(my * m, m), :],
                    dst_ref=out_ref.at[pl.ds(my * m, m), :],
                    send_sem=send_sems.at[3 + off - 1],
                    recv_sem=b_recv_sems.at[3 - off],
                    device_id=(t,),
                    device_id_type=MESH_ID,
                )
                rdma.start()
                sends.append(rdma)
            for j in range(3):
                src_dev = (my + 1 + j) % N_DEV
                recv = pltpu.make_async_remote_copy(
                    src_ref=x_ref.at[pl.ds(0, m), :],
                    dst_ref=out_ref.at[pl.ds(src_dev * m, m), :],
                    send_sem=send_sems.at[j],
                    recv_sem=b_recv_sems.at[j],
                    device_id=(src_dev,),
                    device_id_type=MESH_ID,
                )
                recv.wait_recv()
        for rdma in sends:
            rdma.wait_send()

    out_rows = m if scatter_only else M
    return pl.pallas_call(
        body,
        out_shape=jax.ShapeDtypeStruct((out_rows, n), partial.dtype),
        in_specs=[pl.BlockSpec(memory_space=pltpu.VMEM)],
        out_specs=pl.BlockSpec(memory_space=pltpu.VMEM),
        scratch_shapes=[
            pltpu.VMEM((3, m, n), partial.dtype),
            pltpu.SemaphoreType.DMA((3,)),
            pltpu.SemaphoreType.DMA((3,)),
            pltpu.SemaphoreType.DMA((6,)),
        ],
        compiler_params=pltpu.CompilerParams(collective_id=collective_id),
    )(partial)


def _mlp_layer(x_full, Win, Wout):
    B, d = x_full.shape
    hl = Win.shape[1]
    n_blk = hl // HID_BLK

    def body(x_ref, win_ref, wout_ref, out_ref):
        @pl.when(pl.program_id(0) == 0)
        def _():
            out_ref[:, :] = jnp.zeros_like(out_ref)

        h = jnp.maximum(
            jnp.dot(x_ref[:, :], win_ref[:, :],
                    preferred_element_type=jnp.float32),
            0.0,
        )
        out_ref[:, :] += jnp.dot(
            h, wout_ref[:, :], preferred_element_type=jnp.float32
        )

    return pl.pallas_call(
        body,
        grid=(n_blk,),
        in_specs=[
            pl.BlockSpec((B, d), lambda c: (0, 0)),
            pl.BlockSpec((d, HID_BLK), lambda c: (0, c)),
            pl.BlockSpec((HID_BLK, d), lambda c: (c, 0)),
        ],
        out_specs=pl.BlockSpec((B, d), lambda c: (0, 0)),
        out_shape=jax.ShapeDtypeStruct((B, d), jnp.float32),
    )(x_full, Win, Wout)


def kernel(x, Win0, Wout0, Win1, Wout1, Win2, Wout2):
    xg = _direct_allgather(x, collective_id=0)
    p0 = _mlp_layer(xg, Win0, Wout0)
    x1 = _direct_allreduce(p0, collective_id=1)
    p1 = _mlp_layer(x1, Win1, Wout1)
    x2 = _direct_allreduce(p1, collective_id=2)
    p2 = _mlp_layer(x2, Win2, Wout2)
    return _direct_allreduce(p2, collective_id=3, scatter_only=True)


# device time: 119125 ns/iter; 1.7714x vs baseline; 1.3432x over previous
import jax
import jax.numpy as jnp
from jax import lax
from jax.experimental import pallas as pl
from jax.experimental.pallas import tpu as pltpu

N_DEV = 4
AXIS = "i"
MESH_ID = pl.DeviceIdType.MESH

B = 256
D = 2048
HL = 4096
GC = D // N_DEV
GR = B // N_DEV


def kernel(x, Win0, Wout0, Win1, Wout1, Win2, Wout2):
    def body(x_ref, win0_ref, wout0_ref, win1_ref, wout1_ref,
             win2_ref, wout2_ref, out_ref,
             xf, h, part, acc_c, acc_r, stage_win, stage_wout,
             ag_recv, p_recv, b_recv, row_recv, send_sems,
             win_sems, wout_sems):
        my = lax.axis_index(AXIS)
        win_refs = [win0_ref, win1_ref, win2_ref]
        wout_refs = [wout0_ref, wout1_ref, wout2_ref]

        send_ctr = [0]
        pending = []

        def rsend(src, dst, recv_sem, dev):
            i = send_ctr[0]
            send_ctr[0] += 1
            rdma = pltpu.make_async_remote_copy(
                src_ref=src, dst_ref=dst,
                send_sem=send_sems.at[i], recv_sem=recv_sem,
                device_id=(dev,), device_id_type=MESH_ID,
            )
            rdma.start()
            pending.append(rdma)

        def rwait(dst, recv_sem):
            pltpu.make_async_remote_copy(
                src_ref=dst, dst_ref=dst,
                send_sem=send_sems.at[0], recv_sem=recv_sem,
                device_id=(my,), device_id_type=MESH_ID,
            ).wait_recv()

        def issue_win(k, g, slot):
            pltpu.make_async_copy(
                win_refs[k].at[pl.ds(g * GC, GC), :],
                stage_win.at[slot], win_sems.at[slot]).start()

        def issue_wout(k, g, slot):
            pltpu.make_async_copy(
                wout_refs[k].at[:, pl.ds(g * GC, GC)],
                stage_wout.at[slot], wout_sems.at[slot]).start()

        def wait_local(sem, ref):
            pltpu.make_async_copy(ref, ref, sem).wait()

        s2_g0 = [(my + 1) % N_DEV, (my + 2) % N_DEV, (my + 3) % N_DEV, my]
        issue_win(0, 0, 0)
        issue_win(0, 1, 1)
        issue_wout(0, s2_g0[0], 0)
        issue_wout(0, s2_g0[1], 1)

        barrier = pltpu.get_barrier_semaphore()
        for off in (1, 2, 3):
            pl.semaphore_signal(barrier, inc=1,
                                device_id=((my + off) % N_DEV,),
                                device_id_type=MESH_ID)
        pl.semaphore_wait(barrier, N_DEV - 1)

        xf[pl.ds(my * GR, GR), :] = x_ref[:, :]
        for off in (1, 2, 3):
            t = (my + off) % N_DEV
            rsend(x_ref, xf.at[pl.ds(my * GR, GR), :],
                  ag_recv.at[3 - off], t)
        for j in range(3):
            rwait(xf.at[pl.ds(((my + 1 + j) % N_DEV) * GR, GR), :],
                  ag_recv.at[j])

        s1_groups = [my, (my + 1) % N_DEV, (my + 3) % N_DEV,
                     (my + 2) % N_DEV]
        s1_slots = [None, 0, 2, 1]
        s2_groups = [(my + 1) % N_DEV, (my + 2) % N_DEV,
                     (my + 3) % N_DEV, my]

        for k in range(3):
            last = k == 2
            wout_groups = s2_groups if not last else [0, 1, 2, 3]
            if k > 0:
                issue_wout(k, wout_groups[0], 0)
                issue_wout(k, wout_groups[1], 1)
            groups = [0, 1, 2, 3] if k == 0 else s1_groups
            for i in range(4):
                g = groups[i]
                if k > 0 and s1_slots[i] is not None:
                    rwait(xf.at[:, pl.ds(g * GC, GC)],
                          b_recv.at[k - 1, s1_slots[i]])
                wait_local(win_sems.at[i % 2], stage_win.at[i % 2])
                contrib = jnp.dot(xf[:, pl.ds(g * GC, GC)],
                                  stage_win[i % 2, :, :],
                                  preferred_element_type=jnp.float32)
                if i == 0:
                    h[:, :] = contrib
                else:
                    h[:, :] += contrib
                if i < 2:
                    issue_win(k, groups[i + 2], i % 2)
            h[:, :] = jnp.maximum(h[:, :], 0.0)

            if k < 2:
                issue_win(k + 1, s1_groups[0], 0)
                issue_win(k + 1, s1_groups[1], 1)
            cols = wout_groups
            for p in range(4):
                t = cols[p]
                wait_local(wout_sems.at[p % 2], stage_wout.at[p % 2])
                part[:, pl.ds(t * GC, GC)] = jnp.dot(
                    h[:, :], stage_wout[p % 2, :, :],
                    preferred_element_type=jnp.float32)
                if p < 2:
                    issue_wout(k, cols[p + 2], p % 2)
                if not last and p < 3:
                    rsend(part.at[:, pl.ds(t * GC, GC)],
                          acc_c.at[2 - p], p_recv.at[k, 2 - p], t)

            if not last:
                for j in range(3):
                    rwait(acc_c.at[j], p_recv.at[k, j])
                xf[:, pl.ds(my * GC, GC)] = (
                    part[:, pl.ds(my * GC, GC)]
                    + acc_c[0, :, :] + acc_c[1, :, :] + acc_c[2, :, :]
                )
                for off, slot in ((3, 0), (1, 2), (2, 1)):
                    t = (my + off) % N_DEV
                    rsend(xf.at[:, pl.ds(my * GC, GC)],
                          xf.at[:, pl.ds(my * GC, GC)],
                          b_recv.at[k, slot], t)
            else:
                for off in (1, 2, 3):
                    t = (my + off) % N_DEV
                    rsend(part.at[pl.ds(t * GR, GR), :],
                          acc_r.at[3 - off], row_recv.at[3 - off], t)
                for j in range(3):
                    rwait(acc_r.at[j], row_recv.at[j])
                out_ref[:, :] = (
                    part[pl.ds(my * GR, GR), :]
                    + acc_r[0, :, :] + acc_r[1, :, :] + acc_r[2, :, :]
                )

        for rdma in pending:
            rdma.wait_send()

    f32 = jnp.float32
    return pl.pallas_call(
        body,
        out_shape=jax.ShapeDtypeStruct((GR, D), f32),
        in_specs=[
            pl.BlockSpec(memory_space=pltpu.VMEM),
            pl.BlockSpec(memory_space=pl.ANY),
            pl.BlockSpec(memory_space=pl.ANY),
            pl.BlockSpec(memory_space=pl.ANY),
            pl.BlockSpec(memory_space=pl.ANY),
            pl.BlockSpec(memory_space=pl.ANY),
            pl.BlockSpec(memory_space=pl.ANY),
        ],
        out_specs=pl.BlockSpec(memory_space=pltpu.VMEM),
        scratch_shapes=[
            pltpu.VMEM((B, D), f32),
            pltpu.VMEM((B, HL), f32),
            pltpu.VMEM((B, D), f32),
            pltpu.VMEM((3, B, GC), f32),
            pltpu.VMEM((3, GR, D), f32),
            pltpu.VMEM((2, GC, HL), f32),
            pltpu.VMEM((2, HL, GC), f32),
            pltpu.SemaphoreType.DMA((3,)),
            pltpu.SemaphoreType.DMA((2, 3)),
            pltpu.SemaphoreType.DMA((2, 3)),
            pltpu.SemaphoreType.DMA((3,)),
            pltpu.SemaphoreType.DMA((18,)),
            pltpu.SemaphoreType.DMA((2,)),
            pltpu.SemaphoreType.DMA((2,)),
        ],
        compiler_params=pltpu.CompilerParams(
            collective_id=0,
            vmem_limit_bytes=120 * 1024 * 1024,
        ),
    )(x, Win0, Wout0, Win1, Wout1, Win2, Wout2)


# device time: 111224 ns/iter; 1.8972x vs baseline; 1.0710x over previous
import jax
import jax.numpy as jnp
from jax import lax
from jax.experimental import pallas as pl
from jax.experimental.pallas import tpu as pltpu

N_DEV = 4
AXIS = "i"
MESH_ID = pl.DeviceIdType.MESH

B = 256
D = 2048
HL = 4096
GC = D // N_DEV
GR = B // N_DEV


def kernel(x, Win0, Wout0, Win1, Wout1, Win2, Wout2):
    def body(x_ref, win0_ref, wout0_ref, win1_ref, wout1_ref,
             win2_ref, wout2_ref, out_ref,
             xf, h, part, acc_c, acc_r, stage_win, stage_wout,
             ag_recv, p_recv, b_recv, row_recv, send_sems,
             win_sems, wout_sems):
        my = lax.axis_index(AXIS)
        win_refs = [win0_ref, win1_ref, win2_ref]
        wout_refs = [wout0_ref, wout1_ref, wout2_ref]

        send_ctr = [0]
        pending = []

        def rsend(src, dst, recv_sem, dev):
            i = send_ctr[0]
            send_ctr[0] += 1
            rdma = pltpu.make_async_remote_copy(
                src_ref=src, dst_ref=dst,
                send_sem=send_sems.at[i], recv_sem=recv_sem,
                device_id=(dev,), device_id_type=MESH_ID,
            )
            rdma.start()
            pending.append(rdma)

        def rwait(dst, recv_sem):
            pltpu.make_async_remote_copy(
                src_ref=dst, dst_ref=dst,
                send_sem=send_sems.at[0], recv_sem=recv_sem,
                device_id=(my,), device_id_type=MESH_ID,
            ).wait_recv()

        def issue_win(k, g, slot):
            pltpu.make_async_copy(
                win_refs[k].at[pl.ds(g * GC, GC), :],
                stage_win.at[slot], win_sems.at[slot]).start()

        def issue_wout(k, g, slot):
            pltpu.make_async_copy(
                wout_refs[k].at[:, pl.ds(g * GC, GC)],
                stage_wout.at[slot], wout_sems.at[slot]).start()

        def wait_local(sem, ref):
            pltpu.make_async_copy(ref, ref, sem).wait()

        s2_g0 = [(my + 1) % N_DEV, (my + 2) % N_DEV, (my + 3) % N_DEV, my]
        issue_win(0, 0, 0)
        issue_win(0, 1, 1)
        issue_wout(0, s2_g0[0], 0)
        issue_wout(0, s2_g0[1], 1)

        barrier = pltpu.get_barrier_semaphore()
        for off in (1, 2, 3):
            pl.semaphore_signal(barrier, inc=1,
                                device_id=((my + off) % N_DEV,),
                                device_id_type=MESH_ID)
        pl.semaphore_wait(barrier, N_DEV - 1)

        xf[pl.ds(my * GR, GR), :] = x_ref[:, :]
        for g in range(4):
            for off in (1, 2, 3):
                t = (my + off) % N_DEV
                rsend(x_ref.at[:, pl.ds(g * GC, GC)],
                      xf.at[pl.ds(my * GR, GR), pl.ds(g * GC, GC)],
                      ag_recv.at[3 - off, g], t)

        s1_groups = [my, (my + 1) % N_DEV, (my + 3) % N_DEV,
                     (my + 2) % N_DEV]
        s1_slots = [None, 0, 2, 1]
        s2_groups = [(my + 1) % N_DEV, (my + 2) % N_DEV,
                     (my + 3) % N_DEV, my]

        for k in range(3):
            last = k == 2
            wout_groups = s2_groups if not last else [0, 1, 2, 3]
            if k > 0:
                issue_wout(k, wout_groups[0], 0)
                issue_wout(k, wout_groups[1], 1)
            groups = [0, 1, 2, 3] if k == 0 else s1_groups
            for i in range(4):
                g = groups[i]
                if k == 0:
                    for j in range(3):
                        rwait(xf.at[pl.ds(((my + 1 + j) % N_DEV) * GR, GR),
                                    pl.ds(g * GC, GC)],
                              ag_recv.at[j, g])
                elif s1_slots[i] is not None:
                    rwait(xf.at[:, pl.ds(g * GC, GC)],
                          b_recv.at[k - 1, s1_slots[i]])
                wait_local(win_sems.at[i % 2], stage_win.at[i % 2])
                contrib = jnp.dot(xf[:, pl.ds(g * GC, GC)],
                                  stage_win[i % 2, :, :],
                                  preferred_element_type=jnp.float32)
                if i == 0:
                    h[:, :] = contrib
                else:
                    h[:, :] += contrib
                if i < 2:
                    issue_win(k, groups[i + 2], i % 2)
            h[:, :] = jnp.maximum(h[:, :], 0.0)

            if k < 2:
                issue_win(k + 1, s1_groups[0], 0)
                issue_win(k + 1, s1_groups[1], 1)
            cols = wout_groups
            for p in range(4):
                t = cols[p]
                wait_local(wout_sems.at[p % 2], stage_wout.at[p % 2])
                part[:, pl.ds(t * GC, GC)] = jnp.dot(
                    h[:, :], stage_wout[p % 2, :, :],
                    preferred_element_type=jnp.float32)
                if p < 2:
                    issue_wout(k, cols[p + 2], p % 2)
                if not last and p < 3:
                    rsend(part.at[:, pl.ds(t * GC, GC)],
                          acc_c.at[2 - p], p_recv.at[k, 2 - p], t)
                if last:
                    for off in (1, 2, 3):
                        rt = (my + off) % N_DEV
                        rsend(part.at[pl.ds(rt * GR, GR),
                                      pl.ds(t * GC, GC)],
                              acc_r.at[3 - off, :, pl.ds(t * GC, GC)],
                              row_recv.at[3 - off, t], rt)

            if not last:
                for j in range(3):
                    rwait(acc_c.at[j], p_recv.at[k, j])
                xf[:, pl.ds(my * GC, GC)] = (
                    part[:, pl.ds(my * GC, GC)]
                    + acc_c[0, :, :] + acc_c[1, :, :] + acc_c[2, :, :]
                )
                for off, slot in ((3, 0), (1, 2), (2, 1)):
                    t = (my + off) % N_DEV
                    rsend(xf.at[:, pl.ds(my * GC, GC)],
                          xf.at[:, pl.ds(my * GC, GC)],
                          b_recv.at[k, slot], t)
            else:
                for j in range(3):
                    for g in range(4):
                        rwait(acc_r.at[j, :, pl.ds(g * GC, GC)],
                              row_recv.at[j, g])
                out_ref[:, :] = (
                    part[pl.ds(my * GR, GR), :]
                    + acc_r[0, :, :] + acc_r[1, :, :] + acc_r[2, :, :]
                )

        for rdma in pending:
            rdma.wait_send()

    f32 = jnp.float32
    return pl.pallas_call(
        body,
        out_shape=jax.ShapeDtypeStruct((GR, D), f32),
        in_specs=[
            pl.BlockSpec(memory_space=pltpu.VMEM),
            pl.BlockSpec(memory_space=pl.ANY),
            pl.BlockSpec(memory_space=pl.ANY),
            pl.BlockSpec(memory_space=pl.ANY),
            pl.BlockSpec(memory_space=pl.ANY),
            pl.BlockSpec(memory_space=pl.ANY),
            pl.BlockSpec(memory_space=pl.ANY),
        ],
        out_specs=pl.BlockSpec(memory_space=pltpu.VMEM),
        scratch_shapes=[
            pltpu.VMEM((B, D), f32),
            pltpu.VMEM((B, HL), f32),
            pltpu.VMEM((B, D), f32),
            pltpu.VMEM((3, B, GC), f32),
            pltpu.VMEM((3, GR, D), f32),
            pltpu.VMEM((2, GC, HL), f32),
            pltpu.VMEM((2, HL, GC), f32),
            pltpu.SemaphoreType.DMA((3, 4)),
            pltpu.SemaphoreType.DMA((2, 3)),
            pltpu.SemaphoreType.DMA((2, 3)),
            pltpu.SemaphoreType.DMA((3, 4)),
            pltpu.SemaphoreType.DMA((36,)),
            pltpu.SemaphoreType.DMA((2,)),
            pltpu.SemaphoreType.DMA((2,)),
        ],
        compiler_params=pltpu.CompilerParams(
            collective_id=0,
            vmem_limit_bytes=120 * 1024 * 1024,
        ),
    )(x, Win0, Wout0, Win1, Wout1, Win2, Wout2)


# device time: 108750 ns/iter; 1.9404x vs baseline; 1.0227x over previous
import jax
import jax.numpy as jnp
from jax import lax
from jax.experimental import pallas as pl
from jax.experimental.pallas import tpu as pltpu

N_DEV = 4
AXIS = "i"
MESH_ID = pl.DeviceIdType.MESH

B = 256
D = 2048
HL = 4096
GC = D // N_DEV
GR = B // N_DEV


def kernel(x, Win0, Wout0, Win1, Wout1, Win2, Wout2):
    def body(x_ref, win0_ref, wout0_ref, win1_ref, wout1_ref,
             win2_ref, wout2_ref, out_ref,
             xf, h, part, acc_c, acc_r, stage_win, stage_wout,
             ag_recv, p_recv, b_recv, row_recv, send_sems,
             win_sems, wout_sems):
        my = lax.axis_index(AXIS)
        win_refs = [win0_ref, win1_ref, win2_ref]
        wout_refs = [wout0_ref, wout1_ref, wout2_ref]

        send_ctr = [0]
        pending = []

        def rsend(src, dst, recv_sem, dev):
            i = send_ctr[0]
            send_ctr[0] += 1
            rdma = pltpu.make_async_remote_copy(
                src_ref=src, dst_ref=dst,
                send_sem=send_sems.at[i], recv_sem=recv_sem,
                device_id=(dev,), device_id_type=MESH_ID,
            )
            rdma.start()
            pending.append(rdma)

        def rwait(dst, recv_sem):
            pltpu.make_async_remote_copy(
                src_ref=dst, dst_ref=dst,
                send_sem=send_sems.at[0], recv_sem=recv_sem,
                device_id=(my,), device_id_type=MESH_ID,
            ).wait_recv()

        def issue_win(k, g, slot):
            pltpu.make_async_copy(
                win_refs[k].at[pl.ds(g * GC, GC), :],
                stage_win.at[slot], win_sems.at[slot]).start()

        def issue_wout(k, g, slot):
            pltpu.make_async_copy(
                wout_refs[k].at[:, pl.ds(g * GC, GC)],
                stage_wout.at[slot], wout_sems.at[slot]).start()

        def wait_local(sem, ref):
            pltpu.make_async_copy(ref, ref, sem).wait()

        s2_g0 = [(my + 1) % N_DEV, (my + 2) % N_DEV, (my + 3) % N_DEV, my]
        issue_win(0, 0, 0)
        issue_win(0, 1, 1)
        issue_wout(0, s2_g0[0], 0)
        issue_wout(0, s2_g0[1], 1)

        barrier = pltpu.get_barrier_semaphore()
        for off in (1, 2, 3):
            pl.semaphore_signal(barrier, inc=1,
                                device_id=((my + off) % N_DEV,),
                                device_id_type=MESH_ID)
        pl.semaphore_wait(barrier, N_DEV - 1)

        xf[pl.ds(my * GR, GR), :] = x_ref[:, :]
        for g in range(4):
            for off in (1, 2, 3):
                t = (my + off) % N_DEV
                rsend(x_ref.at[:, pl.ds(g * GC, GC)],
                      xf.at[pl.ds(my * GR, GR), pl.ds(g * GC, GC)],
                      ag_recv.at[3 - off, g], t)

        s1_groups = [my, (my + 1) % N_DEV, (my + 3) % N_DEV,
                     (my + 2) % N_DEV]
        s1_slots = [None, 0, 2, 1]
        s2_groups = [(my + 1) % N_DEV, (my + 2) % N_DEV,
                     (my + 3) % N_DEV, my]

        for k in range(3):
            last = k == 2
            wout_groups = s2_groups if not last else [0, 1, 2, 3]
            if k > 0:
                issue_wout(k, wout_groups[0], 0)
                issue_wout(k, wout_groups[1], 1)
            groups = [0, 1, 2, 3] if k == 0 else s1_groups
            for i in range(4):
                g = groups[i]
                if k == 0:
                    for j in range(3):
                        rwait(xf.at[pl.ds(((my + 1 + j) % N_DEV) * GR, GR),
                                    pl.ds(g * GC, GC)],
                              ag_recv.at[j, g])
                elif s1_slots[i] is not None:
                    rwait(xf.at[:, pl.ds(g * GC, GC)],
                          b_recv.at[k - 1, s1_slots[i]])
                wait_local(win_sems.at[i % 2], stage_win.at[i % 2])
                contrib = jnp.dot(xf[:, pl.ds(g * GC, GC)],
                                  stage_win[i % 2, :, :],
                                  preferred_element_type=jnp.float32)
                if i == 0:
                    h[:, :] = contrib
                else:
                    h[:, :] += contrib
                if i < 2:
                    issue_win(k, groups[i + 2], i % 2)
            h[:, :] = jnp.maximum(h[:, :], 0.0)

            if k < 2:
                issue_win(k + 1, s1_groups[0], 0)
                issue_win(k + 1, s1_groups[1], 1)
            cols = wout_groups
            for p in range(4):
                t = cols[p]
                wait_local(wout_sems.at[p % 2], stage_wout.at[p % 2])
                part[:, pl.ds(t * GC, GC)] = jnp.dot(
                    h[:, :], stage_wout[p % 2, :, :],
                    preferred_element_type=jnp.float32)
                if p < 2:
                    issue_wout(k, cols[p + 2], p % 2)
                if not last and p < 3:
                    rsend(part.at[:, pl.ds(t * GC, GC)],
                          acc_c.at[2 - p], p_recv.at[k, 2 - p], t)
                if last:
                    for off in (1, 2, 3):
                        rt = (my + off) % N_DEV
                        rsend(part.at[pl.ds(rt * GR, GR),
                                      pl.ds(t * GC, GC)],
                              acc_r.at[3 - off, :, pl.ds(t * GC, GC)],
                              row_recv.at[3 - off, t], rt)

            if not last:
                for j in range(3):
                    rwait(acc_c.at[j], p_recv.at[k, j])
                xf[:, pl.ds(my * GC, GC)] = (
                    part[:, pl.ds(my * GC, GC)]
                    + acc_c[0, :, :] + acc_c[1, :, :] + acc_c[2, :, :]
                )
                for off, slot in ((3, 0), (1, 2), (2, 1)):
                    t = (my + off) % N_DEV
                    rsend(xf.at[:, pl.ds(my * GC, GC)],
                          xf.at[:, pl.ds(my * GC, GC)],
                          b_recv.at[k, slot], t)
            else:
                for j in range(3):
                    for g in range(4):
                        rwait(acc_r.at[j, :, pl.ds(g * GC, GC)],
                              row_recv.at[j, g])
                out_ref[:, :] = (
                    part[pl.ds(my * GR, GR), :]
                    + acc_r[0, :, :] + acc_r[1, :, :] + acc_r[2, :, :]
                )

        for rdma in pending:
            rdma.wait_send()

    f32 = jnp.float32
    return pl.pallas_call(
        body,
        out_shape=jax.ShapeDtypeStruct((GR, D), f32),
        in_specs=[
            pl.BlockSpec(memory_space=pltpu.VMEM),
            pl.BlockSpec(memory_space=pl.ANY),
            pl.BlockSpec(memory_space=pl.ANY),
            pl.BlockSpec(memory_space=pl.ANY),
            pl.BlockSpec(memory_space=pl.ANY),
            pl.BlockSpec(memory_space=pl.ANY),
            pl.BlockSpec(memory_space=pl.ANY),
        ],
        out_specs=pl.BlockSpec(memory_space=pltpu.VMEM),
        scratch_shapes=[
            pltpu.VMEM((B, D), f32),
            pltpu.VMEM((B, HL), f32),
            pltpu.VMEM((B, D), f32),
            pltpu.VMEM((3, B, GC), f32),
            pltpu.VMEM((3, GR, D), f32),
            pltpu.VMEM((2, GC, HL), f32),
            pltpu.VMEM((2, HL, GC), f32),
            pltpu.SemaphoreType.DMA((3, 4)),
            pltpu.SemaphoreType.DMA((2, 3)),
            pltpu.SemaphoreType.DMA((2, 3)),
            pltpu.SemaphoreType.DMA((3, 4)),
            pltpu.SemaphoreType.DMA((36,)),
            pltpu.SemaphoreType.DMA((2,)),
            pltpu.SemaphoreType.DMA((2,)),
        ],
        compiler_params=pltpu.CompilerParams(
            collective_id=0,
            vmem_limit_bytes=120 * 1024 * 1024,
            skip_device_barrier=True,
        ),
    )(x, Win0, Wout0, Win1, Wout1, Win2, Wout2)
